# baseline (speedup 1.0000x reference)
import functools

import jax
import jax.numpy as jnp
import numpy as np
from jax import lax
from jax.experimental import pallas as pl
from jax.experimental.pallas import tpu as pltpu
from jax.experimental.pallas import tpu_sc as plsc

_H = 256
_W = 256
_A = 180
_D = 256
_B = 4
_P = _H * _W

_NC = 2
_NS = 16
_NWORK = _NC * _NS

_KMAX = 10112
_KSTEPS = _KMAX // 128
_SPAN = 16
_GROWS = 200
_IMG_PAD = 66048
_ZCHUNK = _IMG_PAD // _NS
_OCHUNK = _P // _NS

_A_PAD = 192
_PIX_BLK = 512
_N_BLOCKS = _P // _PIX_BLK


def _filter_matrix() -> np.ndarray:
    n = np.arange(_D)
    f = np.zeros(_D, dtype=np.float64)
    f[0] = 0.25
    f[1::2] = -1.0 / (np.pi ** 2 * n[1::2].astype(np.float64) ** 2)
    eye = np.eye(_D, dtype=np.float64)
    C = np.fft.ifft(np.fft.fft(eye, axis=1) * f[None, :], axis=1).real
    C *= np.pi / _A
    return C.astype(np.float32)


def _index_tables():
    angles = np.linspace(0.0, np.pi, _A).astype(np.float32)
    cos = np.cos(angles).astype(np.float32)
    sin = np.sin(angles).astype(np.float32)
    y, x = np.meshgrid(np.arange(_H), np.arange(_W), indexing='ij')
    xc = (x - _W / 2).astype(np.float32)
    yc = (y - _H / 2).astype(np.float32)
    rot = xc[None] * cos[:, None, None] + yc[None] * sin[:, None, None]
    idx = np.clip((rot / (2 * np.pi) * _D).astype(np.int32), 0, _D - 1)
    idx = idx.reshape(_A, _P)

    lmat = np.zeros((_A_PAD, _P), np.int8)
    lmat[:_A] = (idx == 0)
    band = (idx >= 1) & (idx <= 254)

    aa, pp = np.nonzero(band)
    tot = len(aa)
    starts = [round(tot * w / _NWORK) for w in range(_NWORK + 1)]

    lf = np.zeros((_NWORK, _KSTEPS, 128), np.int32)
    pix = np.zeros((_NWORK, _KSTEPS, 128), np.int32)
    a0s = np.zeros(_NWORK, np.int32)
    for w in range(_NWORK):
        s, e = starts[w], starts[w + 1]
        c = e - s
        a0 = int(aa[s] // 8) * 8
        assert c <= _KMAX and int(aa[e - 1]) - a0 < _SPAN
        a0s[w] = a0
        lf[w].reshape(_KMAX)[:c] = (aa[s:e] - a0) * _D + idx[aa[s:e], pp[s:e]]
        pw = pix[w].reshape(_KMAX)
        pw[:c] = pp[s:e]
        pw[c:] = _P + (np.arange(_KMAX - c) % 512)
    return lmat, a0s, lf, pix


_C_MAT = _filter_matrix()
_LMAT, _A0S, _LF, _PIX = _index_tables()


def _filter_body(x_ref, c_ref, f_ref, g_ref):
    fm = jnp.dot(x_ref[...], c_ref[...], preferred_element_type=jnp.float32)
    f_ref[...] = fm
    for b in range(_B):
        blk = fm[b * _A:(b + 1) * _A, :]
        g_ref[b * _GROWS: b * _GROWS + _A, :] = blk - blk[:, 255:256]
        g_ref[b * _GROWS + _A: (b + 1) * _GROWS, :] = jnp.zeros(
            (_GROWS - _A, _D), jnp.float32)


_FILT_CALL = pl.pallas_call(
    _filter_body,
    out_shape=(
        jax.ShapeDtypeStruct((_B * _A, _D), jnp.float32),
        jax.ShapeDtypeStruct((_B * _GROWS, _D), jnp.float32),
    ),
)


def _dense_body(fd0_ref, f255_ref, l_ref, o_ref):
    lmat = l_ref[...].astype(jnp.float32)
    s255 = jnp.sum(f255_ref[...], axis=1, keepdims=True)
    o_ref[...] = s255 + jnp.dot(fd0_ref[...], lmat,
                                preferred_element_type=jnp.float32)


_DENSE_CALL = pl.pallas_call(
    _dense_body,
    grid=(_N_BLOCKS,),
    in_specs=[
        pl.BlockSpec((_B, _A_PAD), lambda i: (0, 0)),
        pl.BlockSpec((_B, _A_PAD), lambda i: (0, 0)),
        pl.BlockSpec((_A_PAD, _PIX_BLK), lambda i: (0, i)),
    ],
    out_specs=pl.BlockSpec((_B, _PIX_BLK), lambda i: (0, i)),
    out_shape=jax.ShapeDtypeStruct((_B, _P), jnp.float32),
)


def _band_body(g_hbm, lf_hbm, pix_hbm, zero_hbm, dummy_hbm,
               out_hbm,
               tbl_v, lf_v, pix_v,
               vals0, vals1, vals2, vals3, sem,
               img0, img1, img2, img3):
    cid = lax.axis_index("c")
    sid = lax.axis_index("s")
    wid = cid * _NS + sid
    imgs = [img0, img1, img2, img3]
    vals = [vals0, vals1, vals2, vals3]

    with jax.named_scope("bp_zero"):
        for b in range(_B):
            pltpu.sync_copy(zero_hbm,
                            imgs[b].at[pl.ds(sid * _ZCHUNK, _ZCHUNK)])

    with jax.named_scope("bp_stage"):
        pltpu.sync_copy(lf_hbm.at[wid], lf_v)
        pltpu.sync_copy(pix_hbm.at[wid], pix_v)

    a0 = jnp.int32(0)
    for w in range(_NWORK):
        a0 = a0 + jnp.where(wid == w, jnp.int32(int(_A0S[w])), jnp.int32(0))

    plsc.subcore_barrier()

    for b in range(_B):
        with jax.named_scope("bp_tbl"):
            off = pl.multiple_of((b * _GROWS + a0) * _D, 2048)
            pltpu.sync_copy(g_hbm.at[pl.ds(off, _SPAN * _D)], tbl_v)
        vals_v = vals[b]

        with jax.named_scope("bp_gather"):
            @plsc.parallel_loop(0, _KSTEPS, unroll=2)
            def gstep(j):
                for l in range(8):
                    iv = lf_v[j, pl.ds(l * 16, 16)]
                    vals_v[j, pl.ds(l * 16, 16)] = plsc.load_gather(
                        tbl_v, [iv])

        with jax.named_scope("bp_scatter"):
            def sstep(j, carry):
                pltpu.async_copy(vals_v.at[j], imgs[b].at[pix_v.at[j]], sem,
                                 add=True)
                return carry

            lax.fori_loop(0, _KSTEPS, sstep, jnp.int32(0))

    with jax.named_scope("bp_drain"):
        for b in range(_B):
            pltpu.make_async_copy(dummy_hbm, vals[b], sem).wait()

    plsc.subcore_barrier()

    with jax.named_scope("bp_out"):
        for b in range(_B):
            pltpu.sync_copy(
                imgs[b].at[pl.ds(sid * _OCHUNK, _OCHUNK)],
                out_hbm.at[pl.ds(cid * (_B * _P) + b * _P + sid * _OCHUNK,
                                 _OCHUNK)])


@functools.cache
def _band_call():
  return pl.kernel(
    _band_body,
    out_type=jax.ShapeDtypeStruct((_NC * _B * _P,), jnp.float32),
    mesh=plsc.VectorSubcoreMesh(core_axis_name="c", subcore_axis_name="s",
                                num_cores=_NC, num_subcores=_NS),
    scratch_types=[
        pltpu.VMEM((_SPAN * _D,), jnp.float32),
        pltpu.VMEM((_KSTEPS, 128), jnp.int32),
        pltpu.VMEM((_KSTEPS, 128), jnp.int32),
        pltpu.VMEM((_KSTEPS, 128), jnp.float32),
        pltpu.VMEM((_KSTEPS, 128), jnp.float32),
        pltpu.VMEM((_KSTEPS, 128), jnp.float32),
        pltpu.VMEM((_KSTEPS, 128), jnp.float32),
        pltpu.SemaphoreType.DMA,
        pltpu.VMEM_SHARED((_IMG_PAD,), jnp.float32),
        pltpu.VMEM_SHARED((_IMG_PAD,), jnp.float32),
        pltpu.VMEM_SHARED((_IMG_PAD,), jnp.float32),
        pltpu.VMEM_SHARED((_IMG_PAD,), jnp.float32),
    ],
    compiler_params=pltpu.CompilerParams(use_tc_tiling_on_sc=False,
                                         needs_layout_passes=False),
  )


@jax.jit
def kernel(sinograms):
    B, A, D = sinograms.shape
    filtered, g = _FILT_CALL(sinograms.reshape(B * A, D), jnp.asarray(_C_MAT))

    f3 = filtered.reshape(B, A, D)
    fd0 = jnp.zeros((B, _A_PAD), jnp.float32).at[:, :A].set(
        f3[:, :, 0] - f3[:, :, 255])
    f255 = jnp.zeros((B, _A_PAD), jnp.float32).at[:, :A].set(f3[:, :, 255])

    dense = _DENSE_CALL(fd0, f255, jnp.asarray(_LMAT))

    band = _band_call()(
        g.reshape(-1),
        jnp.asarray(_LF), jnp.asarray(_PIX),
        jnp.zeros((_ZCHUNK,), jnp.float32),
        jnp.zeros((_KSTEPS, 128), jnp.float32))

    band = band.reshape(_NC, _B, _P)
    rec = (dense + band[0] + band[1]).reshape(B, _H, _W)
    return jnp.clip(rec, 0.0, rec.max())


# speedup vs baseline: 1.4995x; 1.4995x over previous
import functools

import jax
import jax.numpy as jnp
import numpy as np
from jax import lax
from jax.experimental import pallas as pl
from jax.experimental.pallas import tpu as pltpu
from jax.experimental.pallas import tpu_sc as plsc

_H = 256
_W = 256
_A = 180
_D = 256
_B = 4
_P = _H * _W

_NC = 2
_NS = 16
_NWORK = _NC * _NS

_KMAX = 10112
_KSTEPS = _KMAX // 128
_SPAN = 16
_GROWS = 200
_IMG_PAD = 66048
_ZCHUNK = _IMG_PAD // _NS
_OCHUNK = _P // _NS

_A_PAD = 192
_PIX_BLK = 2048
_N_BLOCKS = _P // _PIX_BLK


def _filter_matrix() -> np.ndarray:
    n = np.arange(_D)
    f = np.zeros(_D, dtype=np.float64)
    f[0] = 0.25
    f[1::2] = -1.0 / (np.pi ** 2 * n[1::2].astype(np.float64) ** 2)
    eye = np.eye(_D, dtype=np.float64)
    C = np.fft.ifft(np.fft.fft(eye, axis=1) * f[None, :], axis=1).real
    C *= np.pi / _A
    return C.astype(np.float32)


def _index_tables():
    angles = np.linspace(0.0, np.pi, _A).astype(np.float32)
    cos = np.cos(angles).astype(np.float32)
    sin = np.sin(angles).astype(np.float32)
    y, x = np.meshgrid(np.arange(_H), np.arange(_W), indexing='ij')
    xc = (x - _W / 2).astype(np.float32)
    yc = (y - _H / 2).astype(np.float32)
    rot = xc[None] * cos[:, None, None] + yc[None] * sin[:, None, None]
    idx = np.clip((rot / (2 * np.pi) * _D).astype(np.int32), 0, _D - 1)
    idx = idx.reshape(_A, _P)

    lmat = np.zeros((_A_PAD, _P), np.float16)
    lmat[:_A] = (idx == 0)
    lmat = lmat.astype(jnp.bfloat16)
    band = (idx >= 1) & (idx <= 254)

    aa, pp = np.nonzero(band)
    tot = len(aa)
    starts = [round(tot * w / _NWORK) for w in range(_NWORK + 1)]

    lf = np.zeros((_NWORK, _KSTEPS, 128), np.int32)
    pix = np.zeros((_NWORK, _KSTEPS, 128), np.int32)
    a0s = np.zeros(_NWORK, np.int32)
    for w in range(_NWORK):
        s, e = starts[w], starts[w + 1]
        c = e - s
        a0 = int(aa[s] // 8) * 8
        assert c <= _KMAX and int(aa[e - 1]) - a0 < _SPAN
        a0s[w] = a0
        lf[w].reshape(_KMAX)[:c] = (aa[s:e] - a0) * _D + idx[aa[s:e], pp[s:e]]
        pw = pix[w].reshape(_KMAX)
        pw[:c] = pp[s:e]
        pw[c:] = _P + (np.arange(_KMAX - c) % 512)
    return lmat, a0s, lf, pix


_C_MAT = _filter_matrix()
_LMAT, _A0S, _LF, _PIX = _index_tables()


def _filter_body(x_ref, c_ref, f_ref, g_ref):
    fm = jnp.dot(x_ref[...], c_ref[...], preferred_element_type=jnp.float32)
    f_ref[...] = fm
    for b in range(_B):
        blk = fm[b * _A:(b + 1) * _A, :]
        g_ref[b * _GROWS: b * _GROWS + _A, :] = blk - blk[:, 255:256]
        g_ref[b * _GROWS + _A: (b + 1) * _GROWS, :] = jnp.zeros(
            (_GROWS - _A, _D), jnp.float32)


_FILT_CALL = pl.pallas_call(
    _filter_body,
    out_shape=(
        jax.ShapeDtypeStruct((_B * _A, _D), jnp.float32),
        jax.ShapeDtypeStruct((_B * _GROWS, _D), jnp.float32),
    ),
)


def _dense_body(fd0_ref, f255_ref, l_ref, o_ref):
    lmat = l_ref[...].astype(jnp.float32)

    s255 = jnp.sum(f255_ref[...], axis=1, keepdims=True)
    o_ref[...] = s255 + jnp.dot(fd0_ref[...], lmat,
                                preferred_element_type=jnp.float32)


_DENSE_CALL = pl.pallas_call(
    _dense_body,
    grid=(_N_BLOCKS,),
    in_specs=[
        pl.BlockSpec((_B, _A_PAD), lambda i: (0, 0)),
        pl.BlockSpec((_B, _A_PAD), lambda i: (0, 0)),
        pl.BlockSpec((_A_PAD, _PIX_BLK), lambda i: (0, i)),
    ],
    out_specs=pl.BlockSpec((_B, _PIX_BLK), lambda i: (0, i)),
    out_shape=jax.ShapeDtypeStruct((_B, _P), jnp.float32),
)


def _band_body(g_hbm, lf_hbm, pix_hbm, zero_hbm, dummy_hbm,
               out_hbm,
               tbl_v, lf_v, pix_v,
               vals0, vals1, vals2, vals3, sem,
               img0, img1, img2, img3):
    cid = lax.axis_index("c")
    sid = lax.axis_index("s")
    wid = cid * _NS + sid
    imgs = [img0, img1, img2, img3]
    vals = [vals0, vals1, vals2, vals3]

    with jax.named_scope("bp_zero"):
        for b in range(_B):
            pltpu.sync_copy(zero_hbm,
                            imgs[b].at[pl.ds(sid * _ZCHUNK, _ZCHUNK)])

    with jax.named_scope("bp_stage"):
        pltpu.sync_copy(lf_hbm.at[wid], lf_v)
        pltpu.sync_copy(pix_hbm.at[wid], pix_v)

    a0 = jnp.int32(0)
    for w in range(_NWORK):
        a0 = a0 + jnp.where(wid == w, jnp.int32(int(_A0S[w])), jnp.int32(0))

    plsc.subcore_barrier()

    for b in range(_B):
        with jax.named_scope("bp_tbl"):
            off = pl.multiple_of((b * _GROWS + a0) * _D, 2048)
            pltpu.sync_copy(g_hbm.at[pl.ds(off, _SPAN * _D)], tbl_v)
        vals_v = vals[b]

        with jax.named_scope("bp_gather"):
            @plsc.parallel_loop(0, _KSTEPS, unroll=2)
            def gstep(j):
                for l in range(8):
                    iv = lf_v[j, pl.ds(l * 16, 16)]
                    vals_v[j, pl.ds(l * 16, 16)] = plsc.load_gather(
                        tbl_v, [iv])

        with jax.named_scope("bp_scatter"):
            def sstep(j, carry):
                pltpu.async_copy(vals_v.at[j], imgs[b].at[pix_v.at[j]], sem,
                                 add=True)
                return carry

            lax.fori_loop(0, _KSTEPS, sstep, jnp.int32(0))

    with jax.named_scope("bp_drain"):
        for b in range(_B):
            pltpu.make_async_copy(dummy_hbm, vals[b], sem).wait()

    plsc.subcore_barrier()

    with jax.named_scope("bp_out"):
        for b in range(_B):
            pltpu.sync_copy(
                imgs[b].at[pl.ds(sid * _OCHUNK, _OCHUNK)],
                out_hbm.at[pl.ds(cid * (_B * _P) + b * _P + sid * _OCHUNK,
                                 _OCHUNK)])


@functools.cache
def _band_call():
  return pl.kernel(
    _band_body,
    out_type=jax.ShapeDtypeStruct((_NC * _B * _P,), jnp.float32),
    mesh=plsc.VectorSubcoreMesh(core_axis_name="c", subcore_axis_name="s",
                                num_cores=_NC, num_subcores=_NS),
    scratch_types=[
        pltpu.VMEM((_SPAN * _D,), jnp.float32),
        pltpu.VMEM((_KSTEPS, 128), jnp.int32),
        pltpu.VMEM((_KSTEPS, 128), jnp.int32),
        pltpu.VMEM((_KSTEPS, 128), jnp.float32),
        pltpu.VMEM((_KSTEPS, 128), jnp.float32),
        pltpu.VMEM((_KSTEPS, 128), jnp.float32),
        pltpu.VMEM((_KSTEPS, 128), jnp.float32),
        pltpu.SemaphoreType.DMA,
        pltpu.VMEM_SHARED((_IMG_PAD,), jnp.float32),
        pltpu.VMEM_SHARED((_IMG_PAD,), jnp.float32),
        pltpu.VMEM_SHARED((_IMG_PAD,), jnp.float32),
        pltpu.VMEM_SHARED((_IMG_PAD,), jnp.float32),
    ],
    compiler_params=pltpu.CompilerParams(use_tc_tiling_on_sc=False,
                                         needs_layout_passes=False),
  )


@jax.jit
def kernel(sinograms):
    B, A, D = sinograms.shape
    filtered, g = _FILT_CALL(sinograms.reshape(B * A, D), jnp.asarray(_C_MAT))

    f3 = filtered.reshape(B, A, D)
    fd0 = jnp.zeros((B, _A_PAD), jnp.float32).at[:, :A].set(
        f3[:, :, 0] - f3[:, :, 255])
    f255 = jnp.zeros((B, _A_PAD), jnp.float32).at[:, :A].set(f3[:, :, 255])

    dense = _DENSE_CALL(fd0, f255, jnp.asarray(_LMAT))

    band = _band_call()(
        g.reshape(-1),
        jnp.asarray(_LF), jnp.asarray(_PIX),
        jnp.zeros((_ZCHUNK,), jnp.float32),
        jnp.zeros((_KSTEPS, 128), jnp.float32))

    band = band.reshape(_NC, _B, _P)
    rec = (dense + band[0] + band[1]).reshape(B, _H, _W)
    return jnp.clip(rec, 0.0, rec.max())
